# baseline (speedup 1.0000x reference)
import functools

import jax
import jax.numpy as jnp
from jax import lax
from jax.experimental import pallas as pl
from jax.experimental.pallas import tpu as pltpu
from jax.experimental.pallas import tpu_sc as plsc

_NUM_NUM = 13
_N_CAT = 6
_VOCAB = 100000
_EDIM = 16
_BATCH = 16384
_H1 = 128
_H2 = 64

_NW = 32
_ROWS_PER_W = (_N_CAT * _EDIM) // _NW


def _sc_gather_t(xT, tblT):
    mesh = plsc.VectorSubcoreMesh(core_axis_name="c", subcore_axis_name="s")

    @functools.partial(
        pl.kernel,
        mesh=mesh,
        compiler_params=pltpu.CompilerParams(
            use_tc_tiling_on_sc=True, needs_layout_passes=False
        ),
        out_type=jax.ShapeDtypeStruct((_N_CAT * _EDIM, _BATCH), jnp.float32),
        scratch_types=[
            pltpu.VMEM((_VOCAB,), jnp.float32),
            pltpu.VMEM((_BATCH,), jnp.float32),
        ],
    )
    def k(xT_hbm, tbl_hbm, out_hbm, row_v, buf_v):
        wid = lax.axis_index("s") * 2 + lax.axis_index("c")
        for i in range(_ROWS_PER_W):
            r = wid * _ROWS_PER_W + i
            t = r // _EDIM
            pltpu.sync_copy(tbl_hbm.at[r, :], row_v)
            pltpu.sync_copy(xT_hbm.at[_NUM_NUM + t, :], buf_v)

            def body(j, carry):
                idx = buf_v[pl.ds(j * 16, 16)].astype(jnp.int32)
                buf_v[pl.ds(j * 16, 16)] = plsc.load_gather(row_v, [idx])
                return carry

            lax.fori_loop(0, _BATCH // 16, body, 0)
            pltpu.sync_copy(buf_v, out_hbm.at[r, :])

    return k(xT, tblT)


def _tc_mlp_t(xT, embT, w1a, w1b, b1c, w2, b2c, w3, b3c):
    blk = 2048
    grid = _BATCH // blk
    dn = (((0,), (0,)), ((), ()))

    def body(xt, et, w1a_r, w1b_r, b1_r, w2_r, b2_r, w3_r, b3_r, o):
        xnum = xt[...][:_NUM_NUM, :]
        h = lax.dot_general(w1a_r[...], xnum, dn, preferred_element_type=jnp.float32)
        h = h + lax.dot_general(w1b_r[...], et[...], dn, preferred_element_type=jnp.float32)
        h = jnp.maximum(h + b1_r[...], 0.0)
        h = lax.dot_general(w2_r[...], h, dn, preferred_element_type=jnp.float32) + b2_r[...]
        h = jnp.maximum(h, 0.0)
        o32 = lax.dot_general(w3_r[...], h, dn, preferred_element_type=jnp.float32) + b3_r[...]
        o[...] = jax.nn.sigmoid(o32)

    full = lambda shape: pl.BlockSpec(shape, lambda i: (0, 0))
    return pl.pallas_call(
        body,
        grid=(grid,),
        in_specs=[
            pl.BlockSpec((_NUM_NUM + _N_CAT, blk), lambda i: (0, i)),
            pl.BlockSpec((_N_CAT * _EDIM, blk), lambda i: (0, i)),
            full((_NUM_NUM, _H1)),
            full((_N_CAT * _EDIM, _H1)),
            full((_H1, 1)),
            full((_H1, _H2)),
            full((_H2, 1)),
            full((_H2, 1)),
            full((1, 1)),
        ],
        out_specs=pl.BlockSpec((1, blk), lambda i: (0, i)),
        out_shape=jax.ShapeDtypeStruct((1, _BATCH), jnp.float32),
    )(xT, embT, w1a, w1b, b1c, w2, b2c, w3, b3c)


def kernel(x, tables, w1, b1, w2, b2, w3, b3):
    xT = x.T
    tblT = tables.transpose(0, 2, 1).reshape(_N_CAT * _EDIM, _VOCAB)
    embT = _sc_gather_t(xT, tblT)
    o = _tc_mlp_t(
        xT,
        embT,
        w1[:_NUM_NUM],
        w1[_NUM_NUM:],
        b1.reshape(_H1, 1),
        w2,
        b2.reshape(_H2, 1),
        w3,
        b3.reshape(1, 1),
    )
    return o.reshape(_BATCH, 1)
